# baseline (speedup 1.0000x reference)
import functools

import jax
import jax.numpy as jnp
from jax import lax
from jax.experimental import pallas as pl
from jax.experimental.pallas import tpu as pltpu
from jax.experimental.pallas import tpu_sc as plsc

N = 1_600_000
B = 1024
D = 8
NUM_CLASSES = 10
NC = 2
NS = 16
NW = NC * NS
NBLK = N // 128
BPW = 400
KBLK = 20
ACC_HALF = B * D
LSTRIDE = 4 * B


def _sc_partials(x3d, batch):
    mesh = plsc.VectorSubcoreMesh(core_axis_name="c", subcore_axis_name="s")

    @functools.partial(
        pl.kernel,
        out_type=(
            jax.ShapeDtypeStruct((NW, ACC_HALF), jnp.float32),
            jax.ShapeDtypeStruct((NW, B), jnp.float32),
        ),
        mesh=mesh,
        compiler_params=pltpu.CompilerParams(needs_layout_passes=False),
        scratch_types=[
            pltpu.VMEM((KBLK, 4, 128), jnp.float32),
            pltpu.VMEM((KBLK * 128,), jnp.int32),
            pltpu.VMEM((16 * 4 * B,), jnp.float32),
            pltpu.VMEM((16 * B,), jnp.float32),
            pltpu.VMEM((4 * B,), jnp.float32),
            pltpu.VMEM((B,), jnp.float32),
        ],
    )
    def k(x_hbm, b_hbm, out_s, out_c, xbuf, bbuf, acc, cnt, stage, cout):
        wid = lax.axis_index("s") * NC + lax.axis_index("c")
        b0w = wid * BPW
        nchunk = jnp.where(wid == NW - 1, 100 // KBLK, BPW // KBLK)
        lane = lax.iota(jnp.int32, 16)
        zeros16 = jnp.zeros((16,), jnp.float32)
        ones16 = jnp.ones((16,), jnp.float32)
        cnt_off = lane * B
        lane_off = lane * LSTRIDE

        def zero_cnt(i, _):
            cnt[pl.ds(i * 16, 16)] = zeros16
            return 0
        lax.fori_loop(0, (16 * B) // 16, zero_cnt, 0)

        for p in range(2):
            def zero_acc(i, _):
                acc[pl.ds(i * 16, 16)] = zeros16
                return 0
            lax.fori_loop(0, (16 * 4 * B) // 16, zero_acc, 0)

            def chunk_body(c, _):
                blk0 = b0w + c * KBLK
                pltpu.sync_copy(
                    x_hbm.at[pl.ds(blk0, KBLK), pl.ds(p * 4, 4), :], xbuf)
                pltpu.sync_copy(b_hbm.at[pl.ds(blk0 * 128, KBLK * 128)], bbuf)

                def blk_body(blk, _):
                    boff = blk * 128
                    for l in range(8):
                        bv = bbuf[pl.ds(boff + l * 16, 16)]
                        if p == 0:
                            plsc.addupdate_scatter(cnt, [cnt_off + bv], ones16)
                        for dd in range(4):
                            idx = lane_off + dd * B + bv
                            xv = xbuf[blk, dd, pl.ds(l * 16, 16)]
                            plsc.addupdate_scatter(acc, [idx], xv)
                    return 0
                lax.fori_loop(0, KBLK, blk_body, 0)
                return 0
            lax.fori_loop(0, nchunk, chunk_body, 0)

            def fold_acc(s, _):
                t = s * 16
                v = acc[pl.ds(t, 16)]
                for ln in range(1, 16):
                    v = v + acc[pl.ds(ln * LSTRIDE + t, 16)]
                stage[pl.ds(t, 16)] = v
                return 0
            lax.fori_loop(0, LSTRIDE // 16, fold_acc, 0)
            pltpu.sync_copy(stage, out_s.at[wid, pl.ds(p * LSTRIDE, LSTRIDE)])

        def fold_cnt(s, _):
            t = s * 16
            v = cnt[pl.ds(t, 16)]
            for ln in range(1, 16):
                v = v + cnt[pl.ds(ln * B + t, 16)]
            cout[pl.ds(t, 16)] = v
            return 0
        lax.fori_loop(0, B // 16, fold_cnt, 0)
        pltpu.sync_copy(cout, out_c.at[wid])

    return k(x3d, batch)


def _tc_head_body(s_ref, c_ref, w_ref, b_ref, o_ref):
    sums_t = s_ref[pl.ds(0, D), :]
    for w in range(1, NW):
        sums_t = sums_t + s_ref[pl.ds(w * D, D), :]
    counts = jnp.sum(c_ref[:, :], axis=0, keepdims=True)
    pooled_t = sums_t / counts
    logits = lax.dot_general(
        pooled_t, w_ref[:, :],
        dimension_numbers=(((0,), (1,)), ((), ())),
        preferred_element_type=jnp.float32,
    )
    o_ref[:, :] = logits + b_ref[:, :]


def _tc_head(partial_s, partial_c, W, b2):
    return pl.pallas_call(
        _tc_head_body,
        out_shape=jax.ShapeDtypeStruct((B, NUM_CLASSES), jnp.float32),
    )(partial_s, partial_c, W, b2)


def kernel(x, batch, input_ids, attention_mask, W, b):
    del input_ids, attention_mask
    x3d = x.reshape(NBLK, 128, D).transpose(0, 2, 1)
    ps, pc = _sc_partials(x3d, batch)
    ps = ps.reshape(NW * D, B)
    return _tc_head(ps, pc, W, b.reshape(1, NUM_CLASSES))


# speedup vs baseline: 2.9407x; 2.9407x over previous
import functools

import jax
import jax.numpy as jnp
from jax import lax
from jax.experimental import pallas as pl
from jax.experimental.pallas import tpu as pltpu
from jax.experimental.pallas import tpu_sc as plsc

N = 1_600_000
B = 1024
D = 8
NUM_CLASSES = 10
NC = 2
NS = 16
NW = NC * NS
NBLK = N // 128
BPW = 400
KBLK = 20
ACC = 9 * B


def _sc_partials(x3d, batch):
    mesh = plsc.VectorSubcoreMesh(core_axis_name="c", subcore_axis_name="s")

    @functools.partial(
        pl.kernel,
        out_type=(
            jax.ShapeDtypeStruct((NW, D * B), jnp.float32),
            jax.ShapeDtypeStruct((NW, B), jnp.float32),
        ),
        mesh=mesh,
        compiler_params=pltpu.CompilerParams(needs_layout_passes=False),
        scratch_types=[
            pltpu.VMEM((KBLK, D, 128), jnp.float32),
            pltpu.VMEM((KBLK * 128,), jnp.int32),
            pltpu.VMEM((ACC,), jnp.float32),
        ],
    )
    def k(x_hbm, b_hbm, out_s, out_c, xbuf, bbuf, accmem):
        wid = lax.axis_index("s") * NC + lax.axis_index("c")
        b0w = wid * BPW
        nchunk = jnp.where(wid == NW - 1, 100 // KBLK, BPW // KBLK)
        lane = lax.iota(jnp.int32, 16)
        lane9b = jnp.minimum(lane, 8) * B
        lane_d = jnp.minimum(lane, D - 1)
        m9 = lane < 9
        zeros16 = jnp.zeros((16,), jnp.float32)
        oh = [(lane == d).astype(jnp.float32) for d in range(D)]
        oh8 = (lane == 8).astype(jnp.float32)

        def zero_acc(i, _):
            accmem[pl.ds(i * 16, 16)] = zeros16
            return 0
        lax.fori_loop(0, ACC // 16, zero_acc, 0)

        def flush(cur, cnt, accs):
            vals = cnt * oh8
            for d in range(D):
                vals = vals + jnp.sum(accs[d]) * oh[d]
            plsc.addupdate_scatter(accmem, [lane9b + cur], vals, mask=m9)

        def chunk_body(c, carry):
            blk0 = b0w + c * KBLK
            pltpu.sync_copy(x_hbm.at[pl.ds(blk0, KBLK)], xbuf)
            pltpu.sync_copy(b_hbm.at[pl.ds(blk0 * 128, KBLK * 128)], bbuf)

            def blk_body(blk, carry):
                cur, cnt = carry[0], carry[1]
                accs = list(carry[2:])
                for l in range(8):
                    boff = blk * 128 + l * 16
                    bv = bbuf[pl.ds(boff, 16)]
                    bid0 = bv[0]
                    bid15 = bv[15]

                    def uniform_case(cur, cnt, *accs):
                        def same_seg(cur, cnt, *accs):
                            new = [accs[d] + xbuf[blk, d, pl.ds(l * 16, 16)]
                                   for d in range(D)]
                            return (cur, cnt + 16.0, *new)

                        def new_seg(cur, cnt, *accs):
                            flush(cur, cnt, accs)
                            new = [xbuf[blk, d, pl.ds(l * 16, 16)]
                                   for d in range(D)]
                            return (bid0, jnp.float32(16.0), *new)

                        return lax.cond(bid0 == cur, same_seg, new_seg,
                                        cur, cnt, *accs)

                    def mixed_case(cur, cnt, *accs):
                        flush(cur, cnt, accs)

                        def row(i, _):
                            rid = plsc.load_gather(
                                bbuf, [jnp.broadcast_to(boff + i, (16,))])
                            g = plsc.load_gather(
                                xbuf,
                                [jnp.broadcast_to(blk, (16,)),
                                 lane_d,
                                 jnp.broadcast_to(l * 16 + i, (16,))])
                            vals = jnp.where(lane == 8, jnp.float32(1.0), g)
                            plsc.addupdate_scatter(
                                accmem, [lane9b + rid], vals, mask=m9)
                            return 0
                        lax.fori_loop(0, 16, row, 0)
                        return (bid15, jnp.float32(0.0), *([zeros16] * D))

                    cur, cnt, *accs = lax.cond(
                        bid0 == bid15, uniform_case, mixed_case,
                        cur, cnt, *accs)
                return (cur, cnt, *accs)

            return lax.fori_loop(0, KBLK, blk_body, carry)

        init = (jnp.int32(0), jnp.float32(0.0), *([zeros16] * D))
        fin = lax.fori_loop(0, nchunk, chunk_body, init)
        flush(fin[0], fin[1], list(fin[2:]))
        pltpu.sync_copy(accmem.at[pl.ds(0, D * B)], out_s.at[wid])
        pltpu.sync_copy(accmem.at[pl.ds(D * B, B)], out_c.at[wid])

    return k(x3d, batch)


def _tc_head_body(s_ref, c_ref, w_ref, b_ref, o_ref):
    sums_t = s_ref[pl.ds(0, D), :]
    for w in range(1, NW):
        sums_t = sums_t + s_ref[pl.ds(w * D, D), :]
    counts = jnp.sum(c_ref[:, :], axis=0, keepdims=True)
    pooled_t = sums_t / counts
    logits = lax.dot_general(
        pooled_t, w_ref[:, :],
        dimension_numbers=(((0,), (1,)), ((), ())),
        preferred_element_type=jnp.float32,
    )
    o_ref[:, :] = logits + b_ref[:, :]


def _tc_head(partial_s, partial_c, W, b2):
    return pl.pallas_call(
        _tc_head_body,
        out_shape=jax.ShapeDtypeStruct((B, NUM_CLASSES), jnp.float32),
    )(partial_s, partial_c, W, b2)


def kernel(x, batch, input_ids, attention_mask, W, b):
    del input_ids, attention_mask
    x3d = x.reshape(NBLK, 128, D).transpose(0, 2, 1)
    ps, pc = _sc_partials(x3d, batch)
    ps = ps.reshape(NW * D, B)
    return _tc_head(ps, pc, W, b.reshape(1, NUM_CLASSES))


# speedup vs baseline: 4.4308x; 1.5067x over previous
import functools

import jax
import jax.numpy as jnp
from jax import lax
from jax.experimental import pallas as pl
from jax.experimental.pallas import tpu as pltpu
from jax.experimental.pallas import tpu_sc as plsc

N = 1_600_000
B = 1024
D = 8
NUM_CLASSES = 10
NC = 2
NS = 16
NW = NC * NS
NBLK = N // 128
BPW = 400
KBLK = 20
ACC = 9 * B


def _sc_partials(x3d, batch):
    mesh = plsc.VectorSubcoreMesh(core_axis_name="c", subcore_axis_name="s")

    @functools.partial(
        pl.kernel,
        out_type=(
            jax.ShapeDtypeStruct((NW, D * B), jnp.float32),
            jax.ShapeDtypeStruct((NW, B), jnp.float32),
        ),
        mesh=mesh,
        compiler_params=pltpu.CompilerParams(needs_layout_passes=False),
        scratch_types=[
            pltpu.VMEM((KBLK, D, 128), jnp.float32),
            pltpu.VMEM((KBLK * 128,), jnp.int32),
            pltpu.VMEM((ACC,), jnp.float32),
        ],
    )
    def k(x_hbm, b_hbm, out_s, out_c, xbuf, bbuf, accmem):
        wid = lax.axis_index("s") * NC + lax.axis_index("c")
        b0w = wid * BPW
        nchunk = jnp.where(wid == NW - 1, 100 // KBLK, BPW // KBLK)
        lane = lax.iota(jnp.int32, 16)
        lane9b = jnp.minimum(lane, 8) * B
        lane_d = jnp.minimum(lane, D - 1)
        m9 = lane < 9
        zeros16 = jnp.zeros((16,), jnp.float32)
        oh = [(lane == d).astype(jnp.float32) for d in range(D)]
        oh8 = (lane == 8).astype(jnp.float32)

        def zero_acc(i, _):
            accmem[pl.ds(i * 16, 16)] = zeros16
            return 0
        lax.fori_loop(0, ACC // 16, zero_acc, 0)

        def flush(cur, cnt, accs):
            vals = cnt * oh8
            for d in range(D):
                vals = vals + jnp.sum(accs[d]) * oh[d]
            plsc.addupdate_scatter(accmem, [lane9b + cur], vals, mask=m9)

        def chunk_body(c, carry):
            blk0 = b0w + c * KBLK
            pltpu.sync_copy(x_hbm.at[pl.ds(blk0, KBLK)], xbuf)
            pltpu.sync_copy(b_hbm.at[pl.ds(blk0 * 128, KBLK * 128)], bbuf)

            def blk_body(blk, carry):
                bv0 = bbuf[pl.ds(blk * 128, 16)]
                bvl = bbuf[pl.ds(blk * 128 + 112, 16)]
                bid_first = bv0[0]
                bid_last = bvl[15]

                def block_uniform(cur, cnt, *accs):
                    bs = []
                    for d in range(D):
                        v = [xbuf[blk, d, pl.ds(l * 16, 16)] for l in range(8)]
                        bs.append(((v[0] + v[1]) + (v[2] + v[3]))
                                  + ((v[4] + v[5]) + (v[6] + v[7])))

                    def same_seg(cur, cnt, *accs):
                        return (cur, cnt + 128.0,
                                *[accs[d] + bs[d] for d in range(D)])

                    def new_seg(cur, cnt, *accs):
                        flush(cur, cnt, accs)
                        return (bid_first, jnp.float32(128.0), *bs)

                    return lax.cond(bid_first == cur, same_seg, new_seg,
                                    cur, cnt, *accs)

                def block_groups(cur, cnt, *accs):
                    return _groups_body(blk, cur, cnt, list(accs))

                return lax.cond(bid_first == bid_last,
                                block_uniform, block_groups, *carry)

            def _groups_body(blk, cur, cnt, accs):
                for l in range(8):
                    boff = blk * 128 + l * 16
                    bv = bbuf[pl.ds(boff, 16)]
                    bid0 = bv[0]
                    bid15 = bv[15]

                    def uniform_case(cur, cnt, *accs):
                        def same_seg(cur, cnt, *accs):
                            new = [accs[d] + xbuf[blk, d, pl.ds(l * 16, 16)]
                                   for d in range(D)]
                            return (cur, cnt + 16.0, *new)

                        def new_seg(cur, cnt, *accs):
                            flush(cur, cnt, accs)
                            new = [xbuf[blk, d, pl.ds(l * 16, 16)]
                                   for d in range(D)]
                            return (bid0, jnp.float32(16.0), *new)

                        return lax.cond(bid0 == cur, same_seg, new_seg,
                                        cur, cnt, *accs)

                    def mixed_case(cur, cnt, *accs):
                        flush(cur, cnt, accs)

                        def row(i, _):
                            rid = plsc.load_gather(
                                bbuf, [jnp.broadcast_to(boff + i, (16,))])
                            g = plsc.load_gather(
                                xbuf,
                                [jnp.broadcast_to(blk, (16,)),
                                 lane_d,
                                 jnp.broadcast_to(l * 16 + i, (16,))])
                            vals = jnp.where(lane == 8, jnp.float32(1.0), g)
                            plsc.addupdate_scatter(
                                accmem, [lane9b + rid], vals, mask=m9)
                            return 0
                        lax.fori_loop(0, 16, row, 0)
                        return (bid15, jnp.float32(0.0), *([zeros16] * D))

                    cur, cnt, *accs = lax.cond(
                        bid0 == bid15, uniform_case, mixed_case,
                        cur, cnt, *accs)
                return (cur, cnt, *accs)

            return lax.fori_loop(0, KBLK, blk_body, carry)

        init = (jnp.int32(0), jnp.float32(0.0), *([zeros16] * D))
        fin = lax.fori_loop(0, nchunk, chunk_body, init)
        flush(fin[0], fin[1], list(fin[2:]))
        pltpu.sync_copy(accmem.at[pl.ds(0, D * B)], out_s.at[wid])
        pltpu.sync_copy(accmem.at[pl.ds(D * B, B)], out_c.at[wid])

    return k(x3d, batch)


def _tc_head_body(s_ref, c_ref, w_ref, b_ref, o_ref):
    sums_t = s_ref[pl.ds(0, D), :]
    for w in range(1, NW):
        sums_t = sums_t + s_ref[pl.ds(w * D, D), :]
    counts = jnp.sum(c_ref[:, :], axis=0, keepdims=True)
    pooled_t = sums_t / counts
    logits = lax.dot_general(
        pooled_t, w_ref[:, :],
        dimension_numbers=(((0,), (1,)), ((), ())),
        preferred_element_type=jnp.float32,
    )
    o_ref[:, :] = logits + b_ref[:, :]


def _tc_head(partial_s, partial_c, W, b2):
    return pl.pallas_call(
        _tc_head_body,
        out_shape=jax.ShapeDtypeStruct((B, NUM_CLASSES), jnp.float32),
    )(partial_s, partial_c, W, b2)


def kernel(x, batch, input_ids, attention_mask, W, b):
    del input_ids, attention_mask
    x3d = x.reshape(NBLK, 128, D).transpose(0, 2, 1)
    ps, pc = _sc_partials(x3d, batch)
    ps = ps.reshape(NW * D, B)
    return _tc_head(ps, pc, W, b.reshape(1, NUM_CLASSES))


# speedup vs baseline: 5.9359x; 1.3397x over previous
import functools

import jax
import jax.numpy as jnp
from jax import lax
from jax.experimental import pallas as pl
from jax.experimental.pallas import tpu as pltpu
from jax.experimental.pallas import tpu_sc as plsc

N = 1_600_000
B = 1024
D = 8
NUM_CLASSES = 10
NC = 2
NS = 16
NW = NC * NS
NBLK = N // 128
BPW = 400
KBLK = 25
ACC = 9 * B


def _sc_partials(x3d, batch):
    mesh = plsc.VectorSubcoreMesh(core_axis_name="c", subcore_axis_name="s")

    @functools.partial(
        pl.kernel,
        out_type=(
            jax.ShapeDtypeStruct((NW, D * B), jnp.float32),
            jax.ShapeDtypeStruct((NW, B), jnp.float32),
        ),
        mesh=mesh,
        compiler_params=pltpu.CompilerParams(needs_layout_passes=False),
        scratch_types=[
            pltpu.VMEM((2, KBLK, D, 128), jnp.float32),
            pltpu.VMEM((2, KBLK * 128,), jnp.int32),
            pltpu.VMEM((ACC,), jnp.float32),
            pltpu.SemaphoreType.DMA,
            pltpu.SemaphoreType.DMA,
            pltpu.SemaphoreType.DMA,
            pltpu.SemaphoreType.DMA,
        ],
    )
    def k(x_hbm, b_hbm, out_s, out_c, xbuf, bbuf, accmem, sx0, sx1, sb0, sb1):
        wid = lax.axis_index("s") * NC + lax.axis_index("c")
        b0w = wid * BPW
        nchunk = jnp.where(wid == NW - 1, 100 // KBLK, BPW // KBLK)
        npair = jnp.where(wid == NW - 1, (100 // KBLK) // 2, (BPW // KBLK) // 2)
        lane = lax.iota(jnp.int32, 16)
        lane9b = jnp.minimum(lane, 8) * B
        lane_d = jnp.minimum(lane, D - 1)
        m9 = lane < 9
        zeros16 = jnp.zeros((16,), jnp.float32)
        oh = [(lane == d).astype(jnp.float32) for d in range(D)]
        oh8 = (lane == 8).astype(jnp.float32)
        sems_x = (sx0, sx1)
        sems_b = (sb0, sb1)

        def dma_x(c, buf):
            return pltpu.make_async_copy(
                x_hbm.at[pl.ds(b0w + c * KBLK, KBLK)], xbuf.at[buf],
                sems_x[buf])

        def dma_b(c, buf):
            return pltpu.make_async_copy(
                b_hbm.at[pl.ds((b0w + c * KBLK) * 128, KBLK * 128)],
                bbuf.at[buf], sems_b[buf])

        def zero_acc(i, _):
            accmem[pl.ds(i * 16, 16)] = zeros16
            return 0
        lax.fori_loop(0, ACC // 16, zero_acc, 0)

        def flush(cur, cnt, accs):
            vals = cnt * oh8
            for d in range(D):
                vals = vals + jnp.sum(accs[d]) * oh[d]
            plsc.addupdate_scatter(accmem, [lane9b + cur], vals, mask=m9)

        def process_chunk(buf, carry):
            def blk_body(blk, carry):
                bv0 = bbuf[buf, pl.ds(blk * 128, 16)]
                bvl = bbuf[buf, pl.ds(blk * 128 + 112, 16)]
                bid_first = bv0[0]
                bid_last = bvl[15]

                def block_uniform(cur, cnt, *accs):
                    bs = []
                    for d in range(D):
                        v = [xbuf[buf, blk, d, pl.ds(l * 16, 16)]
                             for l in range(8)]
                        bs.append(((v[0] + v[1]) + (v[2] + v[3]))
                                  + ((v[4] + v[5]) + (v[6] + v[7])))

                    def same_seg(cur, cnt, *accs):
                        return (cur, cnt + 128.0,
                                *[accs[d] + bs[d] for d in range(D)])

                    def new_seg(cur, cnt, *accs):
                        flush(cur, cnt, accs)
                        return (bid_first, jnp.float32(128.0), *bs)

                    return lax.cond(bid_first == cur, same_seg, new_seg,
                                    cur, cnt, *accs)

                def block_groups(cur, cnt, *accs):
                    return _groups_body(buf, blk, cur, cnt, list(accs))

                return lax.cond(bid_first == bid_last,
                                block_uniform, block_groups, *carry)

            return lax.fori_loop(0, KBLK, blk_body, carry)

        def _groups_body(buf, blk, cur, cnt, accs):
            for l in range(8):
                boff = blk * 128 + l * 16
                bv = bbuf[buf, pl.ds(boff, 16)]
                bid0 = bv[0]
                bid15 = bv[15]

                def uniform_case(cur, cnt, *accs):
                    def same_seg(cur, cnt, *accs):
                        new = [accs[d] + xbuf[buf, blk, d, pl.ds(l * 16, 16)]
                               for d in range(D)]
                        return (cur, cnt + 16.0, *new)

                    def new_seg(cur, cnt, *accs):
                        flush(cur, cnt, accs)
                        new = [xbuf[buf, blk, d, pl.ds(l * 16, 16)]
                               for d in range(D)]
                        return (bid0, jnp.float32(16.0), *new)

                    return lax.cond(bid0 == cur, same_seg, new_seg,
                                    cur, cnt, *accs)

                def mixed_case(cur, cnt, *accs):
                    flush(cur, cnt, accs)

                    def row(i, _):
                        rid = plsc.load_gather(
                            bbuf,
                            [jnp.broadcast_to(buf, (16,)),
                             jnp.broadcast_to(boff + i, (16,))])
                        g = plsc.load_gather(
                            xbuf,
                            [jnp.broadcast_to(buf, (16,)),
                             jnp.broadcast_to(blk, (16,)),
                             lane_d,
                             jnp.broadcast_to(l * 16 + i, (16,))])
                        vals = jnp.where(lane == 8, jnp.float32(1.0), g)
                        plsc.addupdate_scatter(
                            accmem, [lane9b + rid], vals, mask=m9)
                        return 0
                    lax.fori_loop(0, 16, row, 0)
                    return (bid15, jnp.float32(0.0), *([zeros16] * D))

                cur, cnt, *accs = lax.cond(
                    bid0 == bid15, uniform_case, mixed_case,
                    cur, cnt, *accs)
            return (cur, cnt, *accs)

        dma_x(0, 0).start()
        dma_b(0, 0).start()

        def pair_body(p, carry):
            for b2 in range(2):
                c = 2 * p + b2
                dma_x(c, b2).wait()
                dma_b(c, b2).wait()
                nxt = c + 1

                @pl.when(nxt < nchunk)
                def _():
                    dma_x(nxt, 1 - b2).start()
                    dma_b(nxt, 1 - b2).start()

                carry = process_chunk(b2, carry)
            return carry

        init = (jnp.int32(0), jnp.float32(0.0), *([zeros16] * D))
        fin = lax.fori_loop(0, npair, pair_body, init)
        flush(fin[0], fin[1], list(fin[2:]))
        pltpu.sync_copy(accmem.at[pl.ds(0, D * B)], out_s.at[wid])
        pltpu.sync_copy(accmem.at[pl.ds(D * B, B)], out_c.at[wid])

    return k(x3d, batch)


def _tc_head_body(s_ref, c_ref, w_ref, b_ref, o_ref):
    sums_t = s_ref[pl.ds(0, D), :]
    for w in range(1, NW):
        sums_t = sums_t + s_ref[pl.ds(w * D, D), :]
    counts = jnp.sum(c_ref[:, :], axis=0, keepdims=True)
    pooled_t = sums_t / counts
    logits = lax.dot_general(
        pooled_t, w_ref[:, :],
        dimension_numbers=(((0,), (1,)), ((), ())),
        preferred_element_type=jnp.float32,
    )
    o_ref[:, :] = logits + b_ref[:, :]


def _tc_head(partial_s, partial_c, W, b2):
    return pl.pallas_call(
        _tc_head_body,
        out_shape=jax.ShapeDtypeStruct((B, NUM_CLASSES), jnp.float32),
    )(partial_s, partial_c, W, b2)


def kernel(x, batch, input_ids, attention_mask, W, b):
    del input_ids, attention_mask
    x3d = x.reshape(NBLK, 128, D).transpose(0, 2, 1)
    ps, pc = _sc_partials(x3d, batch)
    ps = ps.reshape(NW * D, B)
    return _tc_head(ps, pc, W, b.reshape(1, NUM_CLASSES))


# speedup vs baseline: 6.1314x; 1.0329x over previous
import functools

import jax
import jax.numpy as jnp
from jax import lax
from jax.experimental import pallas as pl
from jax.experimental.pallas import tpu as pltpu
from jax.experimental.pallas import tpu_sc as plsc

N = 1_600_000
B = 1024
D = 8
NUM_CLASSES = 10
NC = 2
NS = 16
NW = NC * NS
NBLK = N // 128
BPW = 400
KBLK = 25
ACC = 9 * B


def _sc_partials(x3d, batch):
    mesh = plsc.VectorSubcoreMesh(core_axis_name="c", subcore_axis_name="s")

    @functools.partial(
        pl.kernel,
        out_type=(
            jax.ShapeDtypeStruct((NW, D * B), jnp.float32),
            jax.ShapeDtypeStruct((NW, B), jnp.float32),
        ),
        mesh=mesh,
        compiler_params=pltpu.CompilerParams(needs_layout_passes=False),
        scratch_types=[
            pltpu.VMEM((2, KBLK, D, 128), jnp.float32),
            pltpu.VMEM((2, KBLK * 128,), jnp.int32),
            pltpu.VMEM((ACC,), jnp.float32),
            pltpu.SemaphoreType.DMA,
            pltpu.SemaphoreType.DMA,
            pltpu.SemaphoreType.DMA,
            pltpu.SemaphoreType.DMA,
        ],
    )
    def k(x_hbm, b_hbm, out_s, out_c, xbuf, bbuf, accmem, sx0, sx1, sb0, sb1):
        wid = lax.axis_index("s") * NC + lax.axis_index("c")
        b0w = wid * BPW
        nchunk = jnp.where(wid == NW - 1, 100 // KBLK, BPW // KBLK)
        npair = jnp.where(wid == NW - 1, (100 // KBLK) // 2, (BPW // KBLK) // 2)
        lane = lax.iota(jnp.int32, 16)
        lane9b = jnp.minimum(lane, 8) * B
        lane_d = jnp.minimum(lane, D - 1)
        m9 = lane < 9
        zeros16 = jnp.zeros((16,), jnp.float32)
        oh = [(lane == d).astype(jnp.float32) for d in range(D)]
        oh8 = (lane == 8).astype(jnp.float32)
        sems_x = (sx0, sx1)
        sems_b = (sb0, sb1)

        def dma_x(c, buf):
            return pltpu.make_async_copy(
                x_hbm.at[pl.ds(b0w + c * KBLK, KBLK)], xbuf.at[buf],
                sems_x[buf])

        def dma_b(c, buf):
            return pltpu.make_async_copy(
                b_hbm.at[pl.ds((b0w + c * KBLK) * 128, KBLK * 128)],
                bbuf.at[buf], sems_b[buf])

        def zero_acc(i, _):
            accmem[pl.ds(i * 16, 16)] = zeros16
            return 0
        lax.fori_loop(0, ACC // 16, zero_acc, 0)

        def flush(cur, cnt, accs):
            vals = cnt * oh8
            for d in range(D):
                vals = vals + jnp.sum(accs[d]) * oh[d]
            plsc.addupdate_scatter(accmem, [lane9b + cur], vals, mask=m9)

        def process_chunk(buf, carry):
            def blk_body(blk, carry):
                bv0 = bbuf[buf, pl.ds(blk * 128, 16)]
                bvl = bbuf[buf, pl.ds(blk * 128 + 112, 16)]
                bid_first = bv0[0]
                bid_last = bvl[15]

                def block_uniform(cur, cnt, *accs):
                    bs = []
                    for d in range(D):
                        v = [xbuf[buf, blk, d, pl.ds(l * 16, 16)]
                             for l in range(8)]
                        bs.append(((v[0] + v[1]) + (v[2] + v[3]))
                                  + ((v[4] + v[5]) + (v[6] + v[7])))

                    def same_seg(cur, cnt, *accs):
                        return (cur, cnt + 128.0,
                                *[accs[d] + bs[d] for d in range(D)])

                    def new_seg(cur, cnt, *accs):
                        flush(cur, cnt, accs)
                        return (bid_first, jnp.float32(128.0), *bs)

                    return lax.cond(bid_first == cur, same_seg, new_seg,
                                    cur, cnt, *accs)

                def block_groups(cur, cnt, *accs):
                    return _groups_body(buf, blk, cur, cnt, list(accs))

                return lax.cond(bid_first == bid_last,
                                block_uniform, block_groups, *carry)

            return plsc.parallel_loop(0, KBLK, carry=carry)(blk_body)

        def _groups_body(buf, blk, cur, cnt, accs):
            for l in range(8):
                boff = blk * 128 + l * 16
                bv = bbuf[buf, pl.ds(boff, 16)]
                bid0 = bv[0]
                bid15 = bv[15]

                def uniform_case(cur, cnt, *accs):
                    def same_seg(cur, cnt, *accs):
                        new = [accs[d] + xbuf[buf, blk, d, pl.ds(l * 16, 16)]
                               for d in range(D)]
                        return (cur, cnt + 16.0, *new)

                    def new_seg(cur, cnt, *accs):
                        flush(cur, cnt, accs)
                        new = [xbuf[buf, blk, d, pl.ds(l * 16, 16)]
                               for d in range(D)]
                        return (bid0, jnp.float32(16.0), *new)

                    return lax.cond(bid0 == cur, same_seg, new_seg,
                                    cur, cnt, *accs)

                def mixed_case(cur, cnt, *accs):
                    flush(cur, cnt, accs)

                    def row(i, _):
                        rid = plsc.load_gather(
                            bbuf,
                            [jnp.broadcast_to(buf, (16,)),
                             jnp.broadcast_to(boff + i, (16,))])
                        g = plsc.load_gather(
                            xbuf,
                            [jnp.broadcast_to(buf, (16,)),
                             jnp.broadcast_to(blk, (16,)),
                             lane_d,
                             jnp.broadcast_to(l * 16 + i, (16,))])
                        vals = jnp.where(lane == 8, jnp.float32(1.0), g)
                        plsc.addupdate_scatter(
                            accmem, [lane9b + rid], vals, mask=m9)
                        return 0
                    lax.fori_loop(0, 16, row, 0)
                    return (bid15, jnp.float32(0.0), *([zeros16] * D))

                cur, cnt, *accs = lax.cond(
                    bid0 == bid15, uniform_case, mixed_case,
                    cur, cnt, *accs)
            return (cur, cnt, *accs)

        dma_x(0, 0).start()
        dma_b(0, 0).start()

        def pair_body(p, carry):
            for b2 in range(2):
                c = 2 * p + b2
                dma_x(c, b2).wait()
                dma_b(c, b2).wait()
                nxt = c + 1

                @pl.when(nxt < nchunk)
                def _():
                    dma_x(nxt, 1 - b2).start()
                    dma_b(nxt, 1 - b2).start()

                carry = process_chunk(b2, carry)
            return carry

        init = (jnp.int32(0), jnp.float32(0.0), *([zeros16] * D))
        fin = lax.fori_loop(0, npair, pair_body, init)
        flush(fin[0], fin[1], list(fin[2:]))
        pltpu.sync_copy(accmem.at[pl.ds(0, D * B)], out_s.at[wid])
        pltpu.sync_copy(accmem.at[pl.ds(D * B, B)], out_c.at[wid])

    return k(x3d, batch)


def _tc_head_body(s_ref, c_ref, w_ref, b_ref, o_ref):
    sums_t = s_ref[pl.ds(0, D), :]
    for w in range(1, NW):
        sums_t = sums_t + s_ref[pl.ds(w * D, D), :]
    counts = jnp.sum(c_ref[:, :], axis=0, keepdims=True)
    pooled_t = sums_t / counts
    logits = lax.dot_general(
        pooled_t, w_ref[:, :],
        dimension_numbers=(((0,), (1,)), ((), ())),
        preferred_element_type=jnp.float32,
    )
    o_ref[:, :] = logits + b_ref[:, :]


def _tc_head(partial_s, partial_c, W, b2):
    return pl.pallas_call(
        _tc_head_body,
        out_shape=jax.ShapeDtypeStruct((B, NUM_CLASSES), jnp.float32),
    )(partial_s, partial_c, W, b2)


def kernel(x, batch, input_ids, attention_mask, W, b):
    del input_ids, attention_mask
    x3d = x.reshape(NBLK, 128, D).transpose(0, 2, 1)
    ps, pc = _sc_partials(x3d, batch)
    ps = ps.reshape(NW * D, B)
    return _tc_head(ps, pc, W, b.reshape(1, NUM_CLASSES))
